# baseline (speedup 1.0000x reference)
import jax
import jax.numpy as jnp
import numpy as np
from jax.experimental import pallas as pl
from jax.experimental.pallas import tpu as pltpu

DT = 0.1
EPS = 1e-5
NPAR = 5
H = 128
LAYERS = 6
NX, NY, NZ = 20, 20, 20
N = NX * NY * NZ
PAD = NX * NZ
CHUNK = 1000
NCH = N // CHUNK
DELTAS = (1, -1, NZ, -NZ, NX * NZ, -(NX * NZ))


def _swish(x):
    return x * (1.0 / (1.0 + jnp.exp(-x)))


def _mask_weights():
    rows = np.arange(N)
    k_c, i_c, j_c = rows % NZ, (rows // NZ) % NX, rows // (NX * NZ)
    masks = [k_c < NZ - 1, k_c > 0, i_c < NX - 1, i_c > 0,
             j_c < NY - 1, j_c > 0]
    masks = np.stack([m.astype(np.float32) for m in masks], axis=1)
    deg = masks.sum(axis=1, keepdims=True)
    return np.concatenate([masks / deg, 1.0 / deg, 1.0 / deg], axis=1)


_MW_NP = _mask_weights()


def _dot(a, b):
    return jnp.dot(a, b, preferred_element_type=jnp.float32)


def _mpnn_kernel(xr_ref, mw_ref,
                 We1_ref, be1_ref, We2_ref, be2_ref,
                 Wm1_ref, bm1_ref, Wm2_ref, bm2_ref,
                 Wu1_ref, bu1_ref, Wu2_ref, bu2_ref,
                 Wo1_ref, bo1_ref, Wo2_ref, bo2_ref,
                 out_ref,
                 f_ref, d_ref, spad_ref, xm_ref):
    xr = xr_ref[:]
    g = xr[:, 1:4]
    gmax = jnp.max(g, axis=0, keepdims=True)
    gmin = jnp.min(g, axis=0, keepdims=True)
    one1 = jnp.ones((1, 1), jnp.float32)
    one5 = jnp.ones((1, 5), jnp.float32)
    shift = jnp.concatenate([0.0 * one1, gmin, 0.0 * one5], axis=1)
    scale = jnp.concatenate([one1, 1.0 / (gmax - gmin), one5], axis=1)
    xm_ref[:] = (xr - shift) * scale

    spad_ref[0:PAD, :] = jnp.zeros((PAD, H), jnp.float32)
    spad_ref[PAD + N:, :] = jnp.zeros((PAD, H), jnp.float32)

    def ds_from_f(f, l, x):
        Wm1 = Wm1_ref[l]
        D = (_dot(f, Wm1[0:128, :]) + _dot(x, Wm1[256:265, :])
             + bm1_ref[l])
        S = _dot(f, Wm1[128:256, :]) - _dot(x[:, 0:4], Wm1[256:260, :])
        return D, S

    We1 = We1_ref[:]
    be1 = be1_ref[:]
    We2 = We2_ref[:]
    be2 = be2_ref[:]

    def emb_body(i, _):
        base = i * CHUNK
        x = xm_ref[pl.ds(base, CHUNK), :]
        f = _swish(_dot(x, We1) + be1)
        f = _swish(_dot(f, We2) + be2)
        f_ref[pl.ds(base, CHUNK), :] = f
        D, S = ds_from_f(f, 0, x)
        d_ref[pl.ds(base, CHUNK), :] = D
        spad_ref[pl.ds(base + PAD, CHUNK), :] = S
        return 0

    jax.lax.fori_loop(0, NCH, emb_body, 0)

    for l in range(LAYERS):
        Wm2 = Wm2_ref[l]
        bm2 = bm2_ref[l]
        Wu1 = Wu1_ref[l]
        bu1 = bu1_ref[l]
        Wu2 = Wu2_ref[l]
        bu2 = bu2_ref[l]

        def msg_body(i, carry):
            s_acc, q_acc = carry
            base = i * CHUNK
            D = d_ref[pl.ds(base, CHUNK), :]
            xm = xm_ref[pl.ds(base, CHUNK), :]
            mw = mw_ref[pl.ds(base, CHUNK), :]

            agg = jnp.zeros((CHUNK, H), jnp.float32)
            for t, delta in enumerate(DELTAS):
                Ssh = spad_ref[pl.ds(base + PAD + delta, CHUNK), :]
                msg = _swish(_dot(_swish(D + Ssh), Wm2) + bm2)
                agg = agg + mw[:, t : t + 1] * msg
            f = f_ref[pl.ds(base, CHUNK), :]
            x = xm
            upd = _swish(_dot(f, Wu1[0:128, :]) + _dot(agg, Wu1[128:256, :])
                         + _dot(x[:, 4:9], Wu1[256:261, :]) + bu1)
            f = f + _swish(_dot(upd, Wu2) + bu2)
            f_ref[pl.ds(base, CHUNK), :] = f
            s_acc = s_acc + jnp.sum(f, axis=0, keepdims=True)
            q_acc = q_acc + jnp.sum(f * f, axis=0, keepdims=True)
            return s_acc, q_acc

        zero = jnp.zeros((1, H), jnp.float32)
        s_tot, q_tot = jax.lax.fori_loop(0, NCH, msg_body, (zero, zero))

        mean = s_tot * (1.0 / N)
        var = (q_tot - N * mean * mean) * (1.0 / (N - 1))
        inv_std = 1.0 / jnp.sqrt(var + EPS)

        def norm_body(i, _):
            base = i * CHUNK
            f = (f_ref[pl.ds(base, CHUNK), :] - mean) * inv_std
            f_ref[pl.ds(base, CHUNK), :] = f
            if l + 1 < LAYERS:
                x = xm_ref[pl.ds(base, CHUNK), :]
                D, S = ds_from_f(f, l + 1, x)
                d_ref[pl.ds(base, CHUNK), :] = D
                spad_ref[pl.ds(base + PAD, CHUNK), :] = S
            else:
                hid = _swish(_dot(f, Wo1_ref[:]) + bo1_ref[:])
                diff = _dot(hid, Wo2_ref[:]) + bo2_ref[:]
                u = xm_ref[pl.ds(base, CHUNK), 0:1]
                out_ref[pl.ds(base, CHUNK), :] = u + DT * diff
            return 0

        jax.lax.fori_loop(0, NCH, norm_body, 0)


def kernel(inputs, case_params, mask, grid, W_emb1, b_emb1, W_emb2, b_emb2,
           Wm1, bm1, Wm2, bm2, Wu1, bu1, Wu2, bu2, Wo1, bo1, Wo2, bo2):
    bs, nx, ny, nz, c = inputs.shape
    u = inputs[..., 0:1].reshape(-1, 1)
    xraw = jnp.concatenate(
        [u, grid.reshape(-1, 3), case_params.reshape(-1, NPAR)], axis=1)
    mw = jnp.asarray(_MW_NP)

    out = pl.pallas_call(
        _mpnn_kernel,
        out_shape=jax.ShapeDtypeStruct((N, 1), jnp.float32),
        scratch_shapes=[
            pltpu.VMEM((N, H), jnp.float32),
            pltpu.VMEM((N, H), jnp.float32),
            pltpu.VMEM((N + 2 * PAD, H), jnp.float32),
            pltpu.VMEM((N, 9), jnp.float32),
        ],
    )(xraw, mw,
      W_emb1, b_emb1.reshape(1, H), W_emb2, b_emb2.reshape(1, H),
      Wm1, bm1.reshape(LAYERS, 1, H), Wm2, bm2.reshape(LAYERS, 1, H),
      Wu1, bu1.reshape(LAYERS, 1, H), Wu2, bu2.reshape(LAYERS, 1, H),
      Wo1, bo1.reshape(1, H // 2), Wo2, bo2.reshape(1, 1))

    return out.reshape(bs, nx, ny, nz, 1)


# speedup vs baseline: 1.1248x; 1.1248x over previous
import jax
import jax.numpy as jnp
import numpy as np
from jax.experimental import pallas as pl
from jax.experimental.pallas import tpu as pltpu

DT = 0.1
EPS = 1e-5
NPAR = 5
H = 128
LAYERS = 6
NX, NY, NZ = 20, 20, 20
N = NX * NY * NZ
PAD = NX * NZ
CHUNK = 1000
NCH = N // CHUNK
DELTAS = (1, -1, NZ, -NZ, NX * NZ, -(NX * NZ))


def _swish(x):
    xh = 0.5 * x
    return xh + xh * jnp.tanh(xh)


def _mask_weights():
    rows = np.arange(N)
    k_c, i_c, j_c = rows % NZ, (rows // NZ) % NX, rows // (NX * NZ)
    masks = [k_c < NZ - 1, k_c > 0, i_c < NX - 1, i_c > 0,
             j_c < NY - 1, j_c > 0]
    masks = np.stack([m.astype(np.float32) for m in masks], axis=1)
    deg = masks.sum(axis=1, keepdims=True)
    return np.concatenate([masks / deg, 1.0 / deg, 1.0 / deg], axis=1)


_MW_NP = _mask_weights()


def _dot(a, b):
    return jnp.dot(a, b, preferred_element_type=jnp.float32)


def _mpnn_kernel(xr_ref, mw_ref,
                 We1_ref, be1_ref, We2_ref, be2_ref,
                 Wm1_ref, bm1_ref, Wm2_ref, bm2_ref,
                 Wu1_ref, bu1_ref, Wu2_ref, bu2_ref,
                 Wo1_ref, bo1_ref, Wo2_ref, bo2_ref,
                 out_ref,
                 f_ref, d_ref, spad_ref, xm_ref):
    xr = xr_ref[:]
    g = xr[:, 1:4]
    gmax = jnp.max(g, axis=0, keepdims=True)
    gmin = jnp.min(g, axis=0, keepdims=True)
    one1 = jnp.ones((1, 1), jnp.float32)
    one5 = jnp.ones((1, 5), jnp.float32)
    shift = jnp.concatenate([0.0 * one1, gmin, 0.0 * one5], axis=1)
    scale = jnp.concatenate([one1, 1.0 / (gmax - gmin), one5], axis=1)
    xm_ref[:] = (xr - shift) * scale

    spad_ref[0:PAD, :] = jnp.zeros((PAD, H), jnp.float32)
    spad_ref[PAD + N:, :] = jnp.zeros((PAD, H), jnp.float32)

    def ds_from_f(f, l, x):
        Wm1 = Wm1_ref[l]
        D = (_dot(f, Wm1[0:128, :]) + _dot(x, Wm1[256:265, :])
             + bm1_ref[l])
        S = _dot(f, Wm1[128:256, :]) - _dot(x[:, 0:4], Wm1[256:260, :])
        return D, S

    We1 = We1_ref[:]
    be1 = be1_ref[:]
    We2 = We2_ref[:]
    be2 = be2_ref[:]

    def emb_body(i, _):
        base = i * CHUNK
        x = xm_ref[pl.ds(base, CHUNK), :]
        f = _swish(_dot(x, We1) + be1)
        f = _swish(_dot(f, We2) + be2)
        f_ref[pl.ds(base, CHUNK), :] = f
        D, S = ds_from_f(f, 0, x)
        d_ref[pl.ds(base, CHUNK), :] = D
        spad_ref[pl.ds(base + PAD, CHUNK), :] = S
        return 0

    jax.lax.fori_loop(0, NCH, emb_body, 0)

    for l in range(LAYERS):
        Wm2 = Wm2_ref[l]
        bm2 = bm2_ref[l]
        Wu1 = Wu1_ref[l]
        bu1 = bu1_ref[l]
        Wu2 = Wu2_ref[l]
        bu2 = bu2_ref[l]

        def msg_body(i, carry):
            s_acc, q_acc = carry
            base = i * CHUNK
            D = d_ref[pl.ds(base, CHUNK), :]
            xm = xm_ref[pl.ds(base, CHUNK), :]
            mw = mw_ref[pl.ds(base, CHUNK), :]

            agg = jnp.zeros((CHUNK, H), jnp.float32)
            for t, delta in enumerate(DELTAS):
                Ssh = spad_ref[pl.ds(base + PAD + delta, CHUNK), :]
                msg = _swish(_dot(_swish(D + Ssh), Wm2) + bm2)
                agg = agg + mw[:, t : t + 1] * msg
            f = f_ref[pl.ds(base, CHUNK), :]
            x = xm
            upd = _swish(_dot(f, Wu1[0:128, :]) + _dot(agg, Wu1[128:256, :])
                         + _dot(x[:, 4:9], Wu1[256:261, :]) + bu1)
            f = f + _swish(_dot(upd, Wu2) + bu2)
            f_ref[pl.ds(base, CHUNK), :] = f
            s_acc = s_acc + jnp.sum(f, axis=0, keepdims=True)
            q_acc = q_acc + jnp.sum(f * f, axis=0, keepdims=True)
            return s_acc, q_acc

        zero = jnp.zeros((1, H), jnp.float32)
        s_tot, q_tot = jax.lax.fori_loop(0, NCH, msg_body, (zero, zero))

        mean = s_tot * (1.0 / N)
        var = (q_tot - N * mean * mean) * (1.0 / (N - 1))
        inv_std = 1.0 / jnp.sqrt(var + EPS)

        def norm_body(i, _):
            base = i * CHUNK
            f = (f_ref[pl.ds(base, CHUNK), :] - mean) * inv_std
            f_ref[pl.ds(base, CHUNK), :] = f
            if l + 1 < LAYERS:
                x = xm_ref[pl.ds(base, CHUNK), :]
                D, S = ds_from_f(f, l + 1, x)
                d_ref[pl.ds(base, CHUNK), :] = D
                spad_ref[pl.ds(base + PAD, CHUNK), :] = S
            else:
                hid = _swish(_dot(f, Wo1_ref[:]) + bo1_ref[:])
                diff = _dot(hid, Wo2_ref[:]) + bo2_ref[:]
                u = xm_ref[pl.ds(base, CHUNK), 0:1]
                out_ref[pl.ds(base, CHUNK), :] = u + DT * diff
            return 0

        jax.lax.fori_loop(0, NCH, norm_body, 0)


def kernel(inputs, case_params, mask, grid, W_emb1, b_emb1, W_emb2, b_emb2,
           Wm1, bm1, Wm2, bm2, Wu1, bu1, Wu2, bu2, Wo1, bo1, Wo2, bo2):
    bs, nx, ny, nz, c = inputs.shape
    u = inputs[..., 0:1].reshape(-1, 1)
    xraw = jnp.concatenate(
        [u, grid.reshape(-1, 3), case_params.reshape(-1, NPAR)], axis=1)
    mw = jnp.asarray(_MW_NP)

    out = pl.pallas_call(
        _mpnn_kernel,
        out_shape=jax.ShapeDtypeStruct((N, 1), jnp.float32),
        scratch_shapes=[
            pltpu.VMEM((N, H), jnp.float32),
            pltpu.VMEM((N, H), jnp.float32),
            pltpu.VMEM((N + 2 * PAD, H), jnp.float32),
            pltpu.VMEM((N, 9), jnp.float32),
        ],
    )(xraw, mw,
      W_emb1, b_emb1.reshape(1, H), W_emb2, b_emb2.reshape(1, H),
      Wm1, bm1.reshape(LAYERS, 1, H), Wm2, bm2.reshape(LAYERS, 1, H),
      Wu1, bu1.reshape(LAYERS, 1, H), Wu2, bu2.reshape(LAYERS, 1, H),
      Wo1, bo1.reshape(1, H // 2), Wo2, bo2.reshape(1, 1))

    return out.reshape(bs, nx, ny, nz, 1)
